# baseline (speedup 1.0000x reference)
import jax
import jax.numpy as jnp
from jax import lax
from jax.experimental import pallas as pl
from jax.experimental.pallas import tpu as pltpu
from jax.experimental.pallas import tpu_sc as plsc

NC = 2
NS = 16
NW = NC * NS
LANES = 16

W = 20
E = 128
N_UNITS = 1024 * 20 + 1024
UNITS_PER_WORKER = N_UNITS // NW
CHUNK = 6
N_CHUNKS = UNITS_PER_WORKER // CHUNK
ROWS = CHUNK * W
HALF_CHUNKS = N_CHUNKS // 2
STAGE_UNITS = HALF_CHUNKS * CHUNK
NBUF = 4
IDX_PER_WORKER = N_CHUNKS * ROWS


def _wsum_body(idx_hbm, pos_hbm, table_hbm, out_hbm, idx_v, pos_v, rows_all,
               stage_v, gsem0, gsem1, gsem2, gsem3):
    gsems = [gsem0, gsem1, gsem2, gsem3]
    cid = lax.axis_index("c")
    sid = lax.axis_index("s")
    wid = sid * NC + cid
    pltpu.sync_copy(idx_hbm.at[pl.ds(wid * IDX_PER_WORKER, IDX_PER_WORKER)],
                    idx_v)
    pltpu.sync_copy(pos_hbm, pos_v)
    base = wid * UNITS_PER_WORKER * E

    def gather_start(c, k):
        pltpu.async_copy(table_hbm.at[idx_v.at[pl.ds(c * ROWS, ROWS)]],
                         rows_all.at[k], gsems[k])

    def gather_wait(c, k):
        pltpu.make_async_copy(table_hbm.at[idx_v.at[pl.ds(c * ROWS, ROWS)]],
                              rows_all.at[k], gsems[k]).wait()

    def compute(c, k):
        slot = lax.rem(c, HALF_CHUNKS) * CHUNK * E

        def j_body(j, carry):
            col = pl.ds(j * LANES, LANES)
            for s in range(CHUNK):
                acc = rows_all[k, s * W, col] * pos_v[0, col]
                for w in range(1, W):
                    acc = acc + rows_all[k, s * W + w, col] * pos_v[w, col]
                stage_v[pl.ds(slot + s * E + j * LANES, LANES)] = acc
            return carry

        lax.fori_loop(0, E // LANES, j_body, 0)

    def group_body(g, carry):
        for k in range(NBUF):
            gather_start(NBUF * g + k, k)
        for k in range(NBUF):
            gather_wait(NBUF * g + k, k)
        for k in range(NBUF):
            c = NBUF * g + k
            compute(c, k)

            if k == NBUF - 1:
                @pl.when(c == HALF_CHUNKS - 1)
                def _():
                    pltpu.sync_copy(
                        stage_v, out_hbm.at[pl.ds(base, STAGE_UNITS * E)])

                @pl.when(c == N_CHUNKS - 1)
                def _():
                    pltpu.sync_copy(
                        stage_v,
                        out_hbm.at[pl.ds(base + STAGE_UNITS * E,
                                         STAGE_UNITS * E)])
        return carry

    lax.fori_loop(0, N_CHUNKS // NBUF, group_body, 0)


@jax.jit
def _run(idx_all, pos, table):
    mesh = plsc.VectorSubcoreMesh(core_axis_name="c", subcore_axis_name="s",
                                  num_cores=NC, num_subcores=NS)
    k = pl.kernel(
        _wsum_body,
        out_type=jax.ShapeDtypeStruct((N_UNITS * E,), jnp.float32),
        mesh=mesh,
        scratch_types=[
            pltpu.VMEM((IDX_PER_WORKER,), jnp.int32),
            pltpu.VMEM((W, E), jnp.float32),
            pltpu.VMEM((NBUF, ROWS, E), jnp.float32),
            pltpu.VMEM((STAGE_UNITS * E,), jnp.float32),
            pltpu.SemaphoreType.DMA,
            pltpu.SemaphoreType.DMA,
            pltpu.SemaphoreType.DMA,
            pltpu.SemaphoreType.DMA,
        ],
    )
    return k(idx_all, pos, table)


def kernel(story, query, word_table, pos_embed):
    b, s, w = story.shape
    idx_all = jnp.concatenate(
        [story.reshape(b * s, w), query], axis=0).reshape(-1)
    out = _run(idx_all, pos_embed[:w], word_table)
    out = out.reshape(N_UNITS, E)
    sentence_sum = out[:b * s].reshape(b, s, E)
    query_sum = out[b * s:]
    return sentence_sum, query_sum


# speedup vs baseline: 1.1767x; 1.1767x over previous
import jax
import jax.numpy as jnp
from jax import lax
from jax.experimental import pallas as pl
from jax.experimental.pallas import tpu as pltpu
from jax.experimental.pallas import tpu_sc as plsc

NC = 2
NS = 16
NW = NC * NS
LANES = 16

W = 20
E = 128
N_UNITS = 1024 * 20 + 1024
UNITS_PER_WORKER = N_UNITS // NW
CHUNK = 4
N_CHUNKS = UNITS_PER_WORKER // CHUNK
ROWS = CHUNK * W
HALF_CHUNKS = N_CHUNKS // 2
STAGE_UNITS = HALF_CHUNKS * CHUNK
NBUF = 6
IDX_PER_WORKER = N_CHUNKS * ROWS


def _wsum_body(idx_hbm, pos_hbm, table_hbm, out_hbm, idx_v, pos_v, rows_all,
               stage_v, gsem0, gsem1, gsem2, gsem3, gsem4, gsem5):
    gsems = [gsem0, gsem1, gsem2, gsem3, gsem4, gsem5]
    cid = lax.axis_index("c")
    sid = lax.axis_index("s")
    wid = sid * NC + cid
    pltpu.sync_copy(idx_hbm.at[pl.ds(wid * IDX_PER_WORKER, IDX_PER_WORKER)],
                    idx_v)
    pltpu.sync_copy(pos_hbm, pos_v)
    base = wid * UNITS_PER_WORKER * E

    def gather_start(c, k):
        pltpu.async_copy(table_hbm.at[idx_v.at[pl.ds(c * ROWS, ROWS)]],
                         rows_all.at[k], gsems[k])

    def gather_wait(c, k):
        pltpu.make_async_copy(table_hbm.at[idx_v.at[pl.ds(c * ROWS, ROWS)]],
                              rows_all.at[k], gsems[k]).wait()

    def compute(c, k):
        slot = lax.rem(c, HALF_CHUNKS) * CHUNK
        for j in range(E // LANES):
            col = pl.ds(j * LANES, LANES)

            def w_body(w, accs):
                pv = pos_v[w, col]
                return tuple(accs[s] + rows_all[k, s * W + w, col] * pv
                             for s in range(CHUNK))

            zero = jnp.zeros((LANES,), jnp.float32)
            accs = lax.fori_loop(0, W, w_body,
                                 tuple(zero for _ in range(CHUNK)))
            for s in range(CHUNK):
                stage_v[pl.ds((slot + s) * E + j * LANES, LANES)] = accs[s]

    def group_body(g, carry):
        for k in range(NBUF):
            gather_start(NBUF * g + k, k)
        for k in range(NBUF):
            gather_wait(NBUF * g + k, k)
        for k in range(NBUF):
            c = NBUF * g + k
            compute(c, k)

            if k == NBUF - 1:
                @pl.when(c == HALF_CHUNKS - 1)
                def _():
                    pltpu.sync_copy(
                        stage_v, out_hbm.at[pl.ds(base, STAGE_UNITS * E)])

                @pl.when(c == N_CHUNKS - 1)
                def _():
                    pltpu.sync_copy(
                        stage_v,
                        out_hbm.at[pl.ds(base + STAGE_UNITS * E,
                                         STAGE_UNITS * E)])
        return carry

    lax.fori_loop(0, N_CHUNKS // NBUF, group_body, 0)


@jax.jit
def _run(idx_all, pos, table):
    mesh = plsc.VectorSubcoreMesh(core_axis_name="c", subcore_axis_name="s",
                                  num_cores=NC, num_subcores=NS)
    k = pl.kernel(
        _wsum_body,
        out_type=jax.ShapeDtypeStruct((N_UNITS * E,), jnp.float32),
        mesh=mesh,
        scratch_types=[
            pltpu.VMEM((IDX_PER_WORKER,), jnp.int32),
            pltpu.VMEM((W, E), jnp.float32),
            pltpu.VMEM((NBUF, ROWS, E), jnp.float32),
            pltpu.VMEM((STAGE_UNITS * E,), jnp.float32),
            pltpu.SemaphoreType.DMA,
            pltpu.SemaphoreType.DMA,
            pltpu.SemaphoreType.DMA,
            pltpu.SemaphoreType.DMA,
            pltpu.SemaphoreType.DMA,
            pltpu.SemaphoreType.DMA,
        ],
    )
    return k(idx_all, pos, table)


def kernel(story, query, word_table, pos_embed):
    b, s, w = story.shape
    idx_all = jnp.concatenate(
        [story.reshape(b * s, w), query], axis=0).reshape(-1)
    out = _run(idx_all, pos_embed[:w], word_table)
    out = out.reshape(N_UNITS, E)
    sentence_sum = out[:b * s].reshape(b, s, E)
    query_sum = out[b * s:]
    return sentence_sum, query_sum


# speedup vs baseline: 1.2834x; 1.0907x over previous
import jax
import jax.numpy as jnp
from jax import lax
from jax.experimental import pallas as pl
from jax.experimental.pallas import tpu as pltpu
from jax.experimental.pallas import tpu_sc as plsc

NC = 2
NS = 16
NW = NC * NS
LANES = 16

W = 20
E = 128
N_UNITS = 1024 * 20 + 1024
UNITS_PER_WORKER = N_UNITS // NW
CHUNK = 6
N_CHUNKS = UNITS_PER_WORKER // CHUNK
ROWS = CHUNK * W
HALF_CHUNKS = N_CHUNKS // 2
STAGE_UNITS = HALF_CHUNKS * CHUNK
NBUF = 4
IDX_PER_WORKER = N_CHUNKS * ROWS


def _wsum_body(idx_hbm, pos_hbm, table_hbm, out_hbm, idx_v, pos_v, rows_all,
               stage_v, gsem0, gsem1, gsem2, gsem3):
    gsems = [gsem0, gsem1, gsem2, gsem3]
    cid = lax.axis_index("c")
    sid = lax.axis_index("s")
    wid = sid * NC + cid
    pltpu.sync_copy(idx_hbm.at[pl.ds(wid * IDX_PER_WORKER, IDX_PER_WORKER)],
                    idx_v)
    pltpu.sync_copy(pos_hbm, pos_v)
    base = wid * UNITS_PER_WORKER * E

    def gather_start(c, k):
        pltpu.async_copy(table_hbm.at[idx_v.at[pl.ds(c * ROWS, ROWS)]],
                         rows_all.at[k], gsems[k])

    def gather_wait(c, k):
        pltpu.make_async_copy(table_hbm.at[idx_v.at[pl.ds(c * ROWS, ROWS)]],
                              rows_all.at[k], gsems[k]).wait()

    def compute(c, k):
        slot = lax.rem(c, HALF_CHUNKS) * CHUNK
        for j in range(E // LANES):
            col = pl.ds(j * LANES, LANES)

            def w_body(w, accs):
                pv = pos_v[w, col]
                return tuple(accs[s] + rows_all[k, s * W + w, col] * pv
                             for s in range(CHUNK))

            zero = jnp.zeros((LANES,), jnp.float32)
            accs = lax.fori_loop(0, W, w_body,
                                 tuple(zero for _ in range(CHUNK)))
            for s in range(CHUNK):
                stage_v[pl.ds((slot + s) * E + j * LANES, LANES)] = accs[s]

    def group_body(g, carry):
        for k in range(NBUF):
            gather_start(NBUF * g + k, k)
        for k in range(NBUF):
            gather_wait(NBUF * g + k, k)
        for k in range(NBUF):
            c = NBUF * g + k
            compute(c, k)

            if k == NBUF - 1:
                @pl.when(c == HALF_CHUNKS - 1)
                def _():
                    pltpu.sync_copy(
                        stage_v, out_hbm.at[pl.ds(base, STAGE_UNITS * E)])

                @pl.when(c == N_CHUNKS - 1)
                def _():
                    pltpu.sync_copy(
                        stage_v,
                        out_hbm.at[pl.ds(base + STAGE_UNITS * E,
                                         STAGE_UNITS * E)])
        return carry

    lax.fori_loop(0, N_CHUNKS // NBUF, group_body, 0)


@jax.jit
def _run(idx_all, pos, table):
    mesh = plsc.VectorSubcoreMesh(core_axis_name="c", subcore_axis_name="s",
                                  num_cores=NC, num_subcores=NS)
    k = pl.kernel(
        _wsum_body,
        out_type=jax.ShapeDtypeStruct((N_UNITS * E,), jnp.float32),
        mesh=mesh,
        scratch_types=[
            pltpu.VMEM((IDX_PER_WORKER,), jnp.int32),
            pltpu.VMEM((W, E), jnp.float32),
            pltpu.VMEM((NBUF, ROWS, E), jnp.float32),
            pltpu.VMEM((STAGE_UNITS * E,), jnp.float32),
            pltpu.SemaphoreType.DMA,
            pltpu.SemaphoreType.DMA,
            pltpu.SemaphoreType.DMA,
            pltpu.SemaphoreType.DMA,
        ],
    )
    return k(idx_all, pos, table)


def kernel(story, query, word_table, pos_embed):
    b, s, w = story.shape
    idx_all = jnp.concatenate(
        [story.reshape(b * s, w), query], axis=0).reshape(-1)
    out = _run(idx_all, pos_embed[:w], word_table)
    out = out.reshape(N_UNITS, E)
    sentence_sum = out[:b * s].reshape(b, s, E)
    query_sum = out[b * s:]
    return sentence_sum, query_sum


# speedup vs baseline: 1.2895x; 1.0047x over previous
import jax
import jax.numpy as jnp
from jax import lax
from jax.experimental import pallas as pl
from jax.experimental.pallas import tpu as pltpu
from jax.experimental.pallas import tpu_sc as plsc

NC = 2
NS = 16
NW = NC * NS
LANES = 16

W = 20
E = 128
N_UNITS = 1024 * 20 + 1024
UNITS_PER_WORKER = N_UNITS // NW
CHUNK = 6
N_CHUNKS = UNITS_PER_WORKER // CHUNK
ROWS = CHUNK * W
HALF_CHUNKS = N_CHUNKS // 2
STAGE_UNITS = HALF_CHUNKS * CHUNK
NBUF = 4
IDX_PER_WORKER = N_CHUNKS * ROWS


def _wsum_body(idx_hbm, pos_hbm, table_hbm, out_hbm, idx_v, pos_v, rows_all,
               stage_v, gsem0, gsem1, gsem2, gsem3, osem):
    gsems = [gsem0, gsem1, gsem2, gsem3]
    cid = lax.axis_index("c")
    sid = lax.axis_index("s")
    wid = sid * NC + cid
    pltpu.sync_copy(idx_hbm.at[pl.ds(wid * IDX_PER_WORKER, IDX_PER_WORKER)],
                    idx_v)
    pltpu.sync_copy(pos_hbm, pos_v)
    base = wid * UNITS_PER_WORKER * E

    def gather_start(c, k):
        pltpu.async_copy(table_hbm.at[idx_v.at[pl.ds(c * ROWS, ROWS)]],
                         rows_all.at[k], gsems[k])

    def gather_wait(c, k):
        pltpu.make_async_copy(table_hbm.at[idx_v.at[pl.ds(c * ROWS, ROWS)]],
                              rows_all.at[k], gsems[k]).wait()

    def compute(c, k):
        slot = lax.rem(c, HALF_CHUNKS) * CHUNK
        for j in range(E // LANES):
            col = pl.ds(j * LANES, LANES)

            def w_body(w, accs):
                pv = pos_v[w, col]
                return tuple(accs[s] + rows_all[k, s * W + w, col] * pv
                             for s in range(CHUNK))

            zero = jnp.zeros((LANES,), jnp.float32)
            accs = lax.fori_loop(0, W, w_body,
                                 tuple(zero for _ in range(CHUNK)))
            for s in range(CHUNK):
                stage_v[pl.ds((slot + s) * E + j * LANES, LANES)] = accs[s]

    def group_body(g, carry):
        for k in range(NBUF):
            gather_start(NBUF * g + k, k)
        for k in range(NBUF):
            gather_wait(NBUF * g + k, k)

        @pl.when(g == HALF_CHUNKS // NBUF)
        def _():
            pltpu.make_async_copy(
                stage_v, out_hbm.at[pl.ds(base, STAGE_UNITS * E)],
                osem).wait()

        for k in range(NBUF):
            c = NBUF * g + k
            compute(c, k)

            if k == NBUF - 1:
                @pl.when(c == HALF_CHUNKS - 1)
                def _():
                    pltpu.async_copy(
                        stage_v, out_hbm.at[pl.ds(base, STAGE_UNITS * E)],
                        osem)

                @pl.when(c == N_CHUNKS - 1)
                def _():
                    pltpu.sync_copy(
                        stage_v,
                        out_hbm.at[pl.ds(base + STAGE_UNITS * E,
                                         STAGE_UNITS * E)])
        return carry

    lax.fori_loop(0, N_CHUNKS // NBUF, group_body, 0)


@jax.jit
def _run(idx_all, pos, table):
    mesh = plsc.VectorSubcoreMesh(core_axis_name="c", subcore_axis_name="s",
                                  num_cores=NC, num_subcores=NS)
    k = pl.kernel(
        _wsum_body,
        out_type=jax.ShapeDtypeStruct((N_UNITS * E,), jnp.float32),
        mesh=mesh,
        scratch_types=[
            pltpu.VMEM((IDX_PER_WORKER,), jnp.int32),
            pltpu.VMEM((W, E), jnp.float32),
            pltpu.VMEM((NBUF, ROWS, E), jnp.float32),
            pltpu.VMEM((STAGE_UNITS * E,), jnp.float32),
            pltpu.SemaphoreType.DMA,
            pltpu.SemaphoreType.DMA,
            pltpu.SemaphoreType.DMA,
            pltpu.SemaphoreType.DMA,
            pltpu.SemaphoreType.DMA,
        ],
    )
    return k(idx_all, pos, table)


def kernel(story, query, word_table, pos_embed):
    b, s, w = story.shape
    idx_all = jnp.concatenate(
        [story.reshape(b * s, w), query], axis=0).reshape(-1)
    out = _run(idx_all, pos_embed[:w], word_table)
    out = out.reshape(N_UNITS, E)
    sentence_sum = out[:b * s].reshape(b, s, E)
    query_sum = out[b * s:]
    return sentence_sum, query_sum
